# Initial kernel scaffold; baseline (speedup 1.0000x reference)
import jax
import jax.numpy as jnp
from jax.experimental import pallas as pl


def kernel(inputs):
    raise NotImplementedError("write your pallas kernel here")


# speedup vs baseline: 4.6866x; 4.6866x over previous
import functools

import jax
import jax.numpy as jnp
from jax import lax
from jax.experimental import pallas as pl
from jax.experimental.pallas import tpu as pltpu
from jax.experimental.pallas import tpu_sc as plsc

_EPS = 1e-06
_N_BINS = 64
_ROWS = 8192
_COLS = 512
_N = _ROWS * _COLS

_NC = 2
_NS = 16
_NW = _NC * _NS
_LANES = 16

_PER_W = _N // _NW
_CHUNK = 16384
_NCHUNK = _PER_W // _CHUNK

_LO = float(_EPS)
_HI = float(1.0 - _EPS)
_SCALE = float(_N_BINS)


def _sc_body(x_hbm, out_hbm, inbuf, outbuf, sem_in, sem_out):
    wid = lax.axis_index("s") * _NC + lax.axis_index("c")
    base = wid * _PER_W

    def chunk_body(ci, _):
        off = base + ci * _CHUNK
        pltpu.async_copy(x_hbm.at[pl.ds(off, _CHUNK)], inbuf, sem_in).wait()

        def vec_body(vi, _):
            j = vi * _LANES
            v = inbuf[pl.ds(j, _LANES)]
            v = jnp.minimum(jnp.maximum(v, _LO), _HI) * _SCALE
            outbuf[pl.ds(j, _LANES)] = v.astype(jnp.int32)
            return 0

        lax.fori_loop(0, _CHUNK // _LANES, vec_body, 0, unroll=4)
        pltpu.async_copy(outbuf, out_hbm.at[pl.ds(off, _CHUNK)], sem_out).wait()
        return 0

    lax.fori_loop(0, _NCHUNK, chunk_body, 0)


_mesh = plsc.VectorSubcoreMesh(core_axis_name="c", subcore_axis_name="s")

_tokenize_flat = functools.partial(
    pl.kernel,
    out_type=jax.ShapeDtypeStruct((_N,), jnp.int32),
    mesh=_mesh,
    scratch_types=[
        pltpu.VMEM((_CHUNK,), jnp.float32),
        pltpu.VMEM((_CHUNK,), jnp.int32),
        pltpu.SemaphoreType.DMA,
        pltpu.SemaphoreType.DMA,
    ],
)(_sc_body)


@jax.jit
def kernel(inputs):
    flat = inputs.reshape(_N)
    return _tokenize_flat(flat).reshape(_ROWS, _COLS)


# speedup vs baseline: 5.1224x; 1.0930x over previous
import functools

import jax
import jax.numpy as jnp
from jax import lax
from jax.experimental import pallas as pl
from jax.experimental.pallas import tpu as pltpu
from jax.experimental.pallas import tpu_sc as plsc

_EPS = 1e-06
_N_BINS = 64
_ROWS = 8192
_COLS = 512
_N = _ROWS * _COLS

_NC = 2
_NS = 16
_NW = _NC * _NS
_LANES = 16

_PER_W = _N // _NW
_CHUNK = 16384
_NCHUNK = _PER_W // _CHUNK
_NVEC = _CHUNK // _LANES

_LO = float(_EPS)
_HI = float(1.0 - _EPS)
_SCALE = float(_N_BINS)


def _sc_body(x_hbm, out_hbm, in0, in1, ot0, ot1, si0, si1, so0, so1):
    wid = lax.axis_index("s") * _NC + lax.axis_index("c")
    base = wid * _PER_W
    inbufs = (in0, in1)
    outbufs = (ot0, ot1)
    sin = (si0, si1)
    sout = (so0, so1)

    def start_in(ci, b):
        off = base + ci * _CHUNK
        return pltpu.async_copy(x_hbm.at[pl.ds(off, _CHUNK)], inbufs[b], sin[b])

    def start_out(ci, b):
        off = base + ci * _CHUNK
        return pltpu.async_copy(outbufs[b], out_hbm.at[pl.ds(off, _CHUNK)], sout[b])

    def compute(src, dst):
        def vec_body(vi, _):
            j = vi * _LANES
            v = src[pl.ds(j, _LANES)]
            v = jnp.minimum(jnp.maximum(v, _LO), _HI) * _SCALE
            dst[pl.ds(j, _LANES)] = v.astype(jnp.int32)
            return 0

        lax.fori_loop(0, _NVEC, vec_body, 0, unroll=8)

    h_in = [start_in(0, 0), start_in(1, 1)]
    h_out = [None, None]
    for ci in range(_NCHUNK):
        b = ci % 2
        h_in[b].wait()
        if ci >= 2:
            h_out[b].wait()
        compute(inbufs[b], outbufs[b])
        h_out[b] = start_out(ci, b)
        if ci + 2 < _NCHUNK:
            h_in[b] = start_in(ci + 2, b)
    h_out[0].wait()
    h_out[1].wait()


_mesh = plsc.VectorSubcoreMesh(core_axis_name="c", subcore_axis_name="s")

_tokenize_flat = functools.partial(
    pl.kernel,
    out_type=jax.ShapeDtypeStruct((_N,), jnp.int32),
    mesh=_mesh,
    scratch_types=[
        pltpu.VMEM((_CHUNK,), jnp.float32),
        pltpu.VMEM((_CHUNK,), jnp.float32),
        pltpu.VMEM((_CHUNK,), jnp.int32),
        pltpu.VMEM((_CHUNK,), jnp.int32),
        pltpu.SemaphoreType.DMA,
        pltpu.SemaphoreType.DMA,
        pltpu.SemaphoreType.DMA,
        pltpu.SemaphoreType.DMA,
    ],
)(_sc_body)


@jax.jit
def kernel(inputs):
    flat = inputs.reshape(_N)
    return _tokenize_flat(flat).reshape(_ROWS, _COLS)


# speedup vs baseline: 9.0896x; 1.7745x over previous
import functools

import jax
import jax.numpy as jnp
from jax import lax
from jax.experimental import pallas as pl
from jax.experimental.pallas import tpu as pltpu
from jax.experimental.pallas import tpu_sc as plsc

_EPS = 1e-06
_N_BINS = 64
_ROWS = 8192
_COLS = 512
_N = _ROWS * _COLS

_NC = 2
_NS = 16
_NW = _NC * _NS
_LANES = 16

_PER_W = _N // _NW
_CHUNK = 16384
_NCHUNK = _PER_W // _CHUNK
_NVEC = _CHUNK // _LANES

_LO = float(_EPS)
_HI = float(1.0 - _EPS)
_SCALE = float(_N_BINS)


def _sc_body(x_hbm, out_hbm, in0, in1, ot0, ot1, si0, si1, so0, so1):
    wid = lax.axis_index("s") * _NC + lax.axis_index("c")
    base = wid * _PER_W
    inbufs = (in0, in1)
    outbufs = (ot0, ot1)
    sin = (si0, si1)
    sout = (so0, so1)

    def start_in(ci, b):
        off = base + ci * _CHUNK
        return pltpu.async_copy(x_hbm.at[pl.ds(off, _CHUNK)], inbufs[b], sin[b])

    def start_out(ci, b):
        off = base + ci * _CHUNK
        return pltpu.async_copy(outbufs[b], out_hbm.at[pl.ds(off, _CHUNK)], sout[b])

    def compute(src, dst):
        @plsc.parallel_loop(0, _CHUNK, step=_LANES, unroll=8)
        def vec_body(j):
            v = src[pl.ds(j, _LANES)]
            v = jnp.minimum(jnp.maximum(v, _LO), _HI) * _SCALE
            dst[pl.ds(j, _LANES)] = v.astype(jnp.int32)

    h_in = [start_in(0, 0), start_in(1, 1)]
    h_out = [None, None]
    for ci in range(_NCHUNK):
        b = ci % 2
        h_in[b].wait()
        if ci >= 2:
            h_out[b].wait()
        compute(inbufs[b], outbufs[b])
        h_out[b] = start_out(ci, b)
        if ci + 2 < _NCHUNK:
            h_in[b] = start_in(ci + 2, b)
    h_out[0].wait()
    h_out[1].wait()


_mesh = plsc.VectorSubcoreMesh(core_axis_name="c", subcore_axis_name="s")

_tokenize_flat = functools.partial(
    pl.kernel,
    out_type=jax.ShapeDtypeStruct((_N,), jnp.int32),
    mesh=_mesh,
    scratch_types=[
        pltpu.VMEM((_CHUNK,), jnp.float32),
        pltpu.VMEM((_CHUNK,), jnp.float32),
        pltpu.VMEM((_CHUNK,), jnp.int32),
        pltpu.VMEM((_CHUNK,), jnp.int32),
        pltpu.SemaphoreType.DMA,
        pltpu.SemaphoreType.DMA,
        pltpu.SemaphoreType.DMA,
        pltpu.SemaphoreType.DMA,
    ],
)(_sc_body)


@jax.jit
def kernel(inputs):
    flat = inputs.reshape(_N)
    return _tokenize_flat(flat).reshape(_ROWS, _COLS)


# speedup vs baseline: 17.3818x; 1.9123x over previous
import functools

import jax
import jax.numpy as jnp
from jax import lax
from jax.experimental import pallas as pl
from jax.experimental.pallas import tpu as pltpu
from jax.experimental.pallas import tpu_sc as plsc

_EPS = 1e-06
_N_BINS = 64
_ROWS = 8192
_COLS = 512

_NC = 2
_NS = 16
_NW = _NC * _NS
_LANES = 16

_ROWS_W = _ROWS // _NW
_CROWS = 32
_NCHUNK = _ROWS_W // _CROWS
_CHUNK = _CROWS * _COLS

_LO = float(_EPS)
_HI = float(1.0 - _EPS)
_SCALE = float(_N_BINS)


def _sc_body(x_hbm, out_hbm, in0, in1, ot0, ot1, si0, si1, so0, so1):
    wid = lax.axis_index("s") * _NC + lax.axis_index("c")
    base = wid * _ROWS_W
    inbufs = (in0, in1)
    outbufs = (ot0, ot1)
    sin = (si0, si1)
    sout = (so0, so1)

    def start_in(ci, b):
        r0 = base + ci * _CROWS
        return pltpu.async_copy(x_hbm.at[pl.ds(r0, _CROWS)], inbufs[b], sin[b])

    def start_out(ci, b):
        r0 = base + ci * _CROWS
        return pltpu.async_copy(outbufs[b], out_hbm.at[pl.ds(r0, _CROWS)], sout[b])

    def compute(src, dst):
        @plsc.parallel_loop(0, _CROWS, step=1)
        def row_body(r):
            @plsc.parallel_loop(0, _COLS, step=_LANES, unroll=8)
            def col_body(c):
                v = src[r, pl.ds(c, _LANES)]
                v = jnp.minimum(jnp.maximum(v, _LO), _HI) * _SCALE
                dst[r, pl.ds(c, _LANES)] = v.astype(jnp.int32)

    h_in = [start_in(0, 0), start_in(1, 1)]
    h_out = [None, None]
    for ci in range(_NCHUNK):
        b = ci % 2
        h_in[b].wait()
        if ci >= 2:
            h_out[b].wait()
        compute(inbufs[b], outbufs[b])
        h_out[b] = start_out(ci, b)
        if ci + 2 < _NCHUNK:
            h_in[b] = start_in(ci + 2, b)
    h_out[0].wait()
    h_out[1].wait()


_mesh = plsc.VectorSubcoreMesh(core_axis_name="c", subcore_axis_name="s")

_tokenize = functools.partial(
    pl.kernel,
    out_type=jax.ShapeDtypeStruct((_ROWS, _COLS), jnp.int32),
    mesh=_mesh,
    scratch_types=[
        pltpu.VMEM((_CROWS, _COLS), jnp.float32),
        pltpu.VMEM((_CROWS, _COLS), jnp.float32),
        pltpu.VMEM((_CROWS, _COLS), jnp.int32),
        pltpu.VMEM((_CROWS, _COLS), jnp.int32),
        pltpu.SemaphoreType.DMA,
        pltpu.SemaphoreType.DMA,
        pltpu.SemaphoreType.DMA,
        pltpu.SemaphoreType.DMA,
    ],
)(_sc_body)


@jax.jit
def kernel(inputs):
    return _tokenize(inputs)
